# baseline (speedup 1.0000x reference)
import functools

import jax
import jax.numpy as jnp
from jax import lax
from jax.experimental import pallas as pl
from jax.experimental.pallas import tpu as pltpu
from jax.experimental.pallas import tpu_sc as plsc

N = 10000
E = 320000
D = 128

NC = 2
NS = 16
DH = D // NC

CHUNK = 128
EPW = E // NS
FULL_CHUNKS = EPW // CHUNK
TAIL = EPW - FULL_CHUNKS * CHUNK
RPT = N // NS

NBUF = 4
ROUNDS = FULL_CHUNKS // NBUF


def _mm_body(x_ref, wt_ref, b_ref, y_ref):
    y_ref[...] = (
        jnp.dot(x_ref[...], wt_ref[0], preferred_element_type=jnp.float32)
        + b_ref[0]
    )


_MM_BM = 1000


def _linear(x, wt_split, b_split):
    nb = N // _MM_BM
    return pl.pallas_call(
        _mm_body,
        grid=(NC, nb),
        in_specs=[
            pl.BlockSpec((_MM_BM, D), lambda c, i: (i, 0)),
            pl.BlockSpec((1, D, DH), lambda c, i: (c, 0, 0)),
            pl.BlockSpec((1, 1, DH), lambda c, i: (c, 0, 0)),
        ],
        out_specs=pl.BlockSpec((_MM_BM, DH), lambda c, i: (c * nb + i, 0)),
        out_shape=jax.ShapeDtypeStruct((NC * N, DH), jnp.float32),
    )(x, wt_split, b_split)


@functools.partial(
    pl.kernel,
    mesh=plsc.VectorSubcoreMesh(core_axis_name="c", subcore_axis_name="s"),
    out_type=jax.ShapeDtypeStruct((N, D), jnp.float32),
    compiler_params=pltpu.CompilerParams(use_tc_tiling_on_sc=False),
    scratch_types=[
        pltpu.VMEM((NBUF, CHUNK), jnp.int32),
        pltpu.VMEM((NBUF, CHUNK), jnp.int32),
        pltpu.VMEM((NBUF, CHUNK, DH), jnp.float32),
        pltpu.VMEM((TAIL,), jnp.int32),
        pltpu.VMEM((TAIL,), jnp.int32),
        pltpu.VMEM((TAIL, DH), jnp.float32),
        pltpu.VMEM_SHARED((N, DH), jnp.float32),
        pltpu.VMEM_SHARED((N, DH), jnp.float32),
        pltpu.SemaphoreType.DMA((NBUF,)),
        pltpu.SemaphoreType.DMA((NBUF,)),
        pltpu.SemaphoreType.DMA((NBUF,)),
    ],
)
def _sc_aggregate(y_hbm, src_hbm, dst_hbm, zeros_hbm, out_hbm,
                  sidx, didx, rows, tsidx, tdidx, trow,
                  y_sh, acc_sh, isem, gsem, ssem):
    c = lax.axis_index("c")
    s = lax.axis_index("s")
    ebase = pl.multiple_of(s * EPW, 8)

    def idx_start(i, b):
        off = pl.multiple_of(ebase + i * CHUNK, 8)
        pltpu.async_copy(src_hbm.at[pl.ds(off, CHUNK)], sidx.at[b], isem.at[b])
        pltpu.async_copy(dst_hbm.at[pl.ds(off, CHUNK)], didx.at[b], isem.at[b])

    def idx_wait(i, b):
        off = pl.multiple_of(ebase + i * CHUNK, 8)
        pltpu.make_async_copy(
            src_hbm.at[pl.ds(off, CHUNK)], sidx.at[b], isem.at[b]).wait()
        pltpu.make_async_copy(
            dst_hbm.at[pl.ds(off, CHUNK)], didx.at[b], isem.at[b]).wait()

    for b in range(NBUF):
        idx_start(b, b)

    r0 = s * RPT
    pltpu.sync_copy(y_hbm.at[pl.ds(c * N + r0, RPT)], y_sh.at[pl.ds(r0, RPT)])
    pltpu.sync_copy(zeros_hbm, acc_sh.at[pl.ds(r0, RPT)])
    plsc.subcore_barrier()

    def round_body(r, _):
        outer = r * NBUF
        for b in range(NBUF):
            i = outer + b
            idx_wait(i, b)
            pltpu.async_copy(y_sh.at[sidx.at[b]], rows.at[b], gsem.at[b])
        for b in range(NBUF):
            i = outer + b
            pltpu.make_async_copy(
                y_sh.at[sidx.at[b]], rows.at[b], gsem.at[b]).wait()
            pltpu.async_copy(rows.at[b], acc_sh.at[didx.at[b]],
                             ssem.at[b], add=True)
        for b in range(NBUF):
            i = outer + b
            pltpu.make_async_copy(
                rows.at[b], acc_sh.at[didx.at[b]], ssem.at[b]).wait()

            @pl.when(r < ROUNDS - 1)
            def _():
                idx_start(i + NBUF, b)
        return 0

    lax.fori_loop(0, ROUNDS, round_body, 0)

    toff = pl.multiple_of(ebase + FULL_CHUNKS * CHUNK, 8)
    pltpu.sync_copy(src_hbm.at[pl.ds(toff, TAIL)], tsidx)
    pltpu.sync_copy(dst_hbm.at[pl.ds(toff, TAIL)], tdidx)
    pltpu.sync_copy(y_sh.at[tsidx], trow)
    pltpu.sync_copy(trow, acc_sh.at[tdidx], add=True)

    plsc.subcore_barrier()

    pltpu.sync_copy(acc_sh.at[pl.ds(r0, RPT)],
                    out_hbm.at[pl.ds(r0, RPT), pl.ds(c * DH, DH)])


def kernel(x, edge_index, W, b):
    wt = W.T
    wt_split = jnp.stack([wt[:, :DH], wt[:, DH:]])
    y = _linear(x, wt_split, b.reshape(NC, 1, DH))
    zeros = jnp.zeros((RPT, DH), jnp.float32)
    return _sc_aggregate(y, edge_index[0], edge_index[1], zeros)
